# Initial kernel scaffold; baseline (speedup 1.0000x reference)
import jax
import jax.numpy as jnp
from jax.experimental import pallas as pl


def kernel(frames, params, lengths):
    raise NotImplementedError("write your pallas kernel here")


# speedup vs baseline: 3.4418x; 3.4418x over previous
import functools

import jax
import jax.numpy as jnp
from jax.experimental import pallas as pl
from jax.experimental.pallas import tpu as pltpu

B, T, C, H, W = 32, 256, 3, 32, 32
FEAT = 512
HID = 128
TOP_K = 8
CHW = C * H * W

ROWS_BLK = 1024


def _enc_kernel(x_ref, wenc_ref, benc_ref, wcat_ref, bcat_ref, out_ref):
    x = x_ref[...]
    feats = jnp.dot(x, wenc_ref[...], preferred_element_type=jnp.float32)
    feats = feats + benc_ref[...]
    g = jnp.dot(feats, wcat_ref[...], preferred_element_type=jnp.float32)
    out_ref[...] = g + bcat_ref[...]


def _encode(x2d, wenc_t, benc, wcat, bcat):
    n_rows = x2d.shape[0]
    grid = (n_rows // ROWS_BLK,)
    return pl.pallas_call(
        _enc_kernel,
        grid=grid,
        in_specs=[
            pl.BlockSpec((ROWS_BLK, CHW), lambda k: (k, 0)),
            pl.BlockSpec((CHW, FEAT), lambda k: (0, 0)),
            pl.BlockSpec((1, FEAT), lambda k: (0, 0)),
            pl.BlockSpec((FEAT, 6 * HID), lambda k: (0, 0)),
            pl.BlockSpec((1, 6 * HID), lambda k: (0, 0)),
        ],
        out_specs=pl.BlockSpec((ROWS_BLK, 6 * HID), lambda k: (k, 0)),
        out_shape=jax.ShapeDtypeStruct((n_rows, 6 * HID), jnp.float32),
    )(x2d, wenc_t, benc, wcat, bcat)


SEQ_BLK = 32


def _gru_kernel(gf_ref, gb_ref, whh_ref, bhh_ref, len_ref,
                outf_ref, outb_ref, h_ref):
    k = pl.program_id(0)

    @pl.when(k == 0)
    def _():
        h_ref[...] = jnp.zeros_like(h_ref)

    lengths = len_ref[...]

    def step(i, _):
        h = h_ref[...]
        gi_f = gf_ref[:, i, :]
        gi_b = gb_ref[:, SEQ_BLK - 1 - i, :]
        gh = jnp.dot(h, whh_ref[...], preferred_element_type=jnp.float32)
        gh = gh + bhh_ref[...]

        t_f = k * SEQ_BLK + i
        t_b = T - 1 - t_f

        def gru_dir(gi, gh_d, h_d, t):
            r = jax.nn.sigmoid(gi[:, :HID] + gh_d[:, :HID])
            z = jax.nn.sigmoid(gi[:, HID:2 * HID] + gh_d[:, HID:2 * HID])
            n = jnp.tanh(gi[:, 2 * HID:] + r * gh_d[:, 2 * HID:])
            h_new = (1.0 - z) * n + z * h_d
            valid = lengths > t
            return jnp.where(valid, h_new, h_d)

        h_f = gru_dir(gi_f, gh[:, :3 * HID], h[:, :HID], t_f)
        h_b = gru_dir(gi_b, gh[:, 3 * HID:], h[:, HID:], t_b)
        outf_ref[:, i, :] = h_f
        outb_ref[:, SEQ_BLK - 1 - i, :] = h_b
        h_ref[...] = jnp.concatenate([h_f, h_b], axis=1)
        return 0

    jax.lax.fori_loop(0, SEQ_BLK, step, 0, unroll=True)


def _gru(gcat3, whh_big, bhh, lengths_col):
    nb = T // SEQ_BLK
    out = pl.pallas_call(
        _gru_kernel,
        grid=(nb,),
        in_specs=[
            pl.BlockSpec((B, SEQ_BLK, 3 * HID), lambda k: (0, k, 0)),
            pl.BlockSpec((B, SEQ_BLK, 3 * HID), lambda k, nb=nb: (0, nb - 1 - k, 1)),
            pl.BlockSpec((2 * HID, 6 * HID), lambda k: (0, 0)),
            pl.BlockSpec((1, 6 * HID), lambda k: (0, 0)),
            pl.BlockSpec((B, 1), lambda k: (0, 0)),
        ],
        out_specs=[
            pl.BlockSpec((B, SEQ_BLK, HID), lambda k: (0, k, 0)),
            pl.BlockSpec((B, SEQ_BLK, HID), lambda k, nb=nb: (0, nb - 1 - k, 0)),
        ],
        out_shape=[
            jax.ShapeDtypeStruct((B, T, HID), jnp.float32),
            jax.ShapeDtypeStruct((B, T, HID), jnp.float32),
        ],
        scratch_shapes=[pltpu.VMEM((B, 2 * HID), jnp.float32)],
        compiler_params=pltpu.CompilerParams(
            dimension_semantics=("arbitrary",)),
    )(gcat3, gcat3, whh_big, bhh, lengths_col)
    out_f, out_b = out
    return out_f, out_b


def _head_kernel(xf_ref, xb_ref, w1f_ref, w1b_ref, b1_ref, w2_ref,
                 len_ref, temp_ref, wto_ref, bto_ref, out_ref):
    xf = xf_ref[...]
    xb = xb_ref[...]
    h1 = jnp.dot(xf.reshape(B * T, HID), w1f_ref[...],
                 preferred_element_type=jnp.float32)
    h1 = h1 + jnp.dot(xb.reshape(B * T, HID), w1b_ref[...],
                      preferred_element_type=jnp.float32)
    h1 = jax.nn.relu(h1 + b1_ref[...])
    scores = jnp.sum(h1.reshape(B, T, 64) * w2_ref[...], axis=2)

    lengths = len_ref[...]
    tpos = jax.lax.broadcasted_iota(jnp.int32, (B, T), 1)
    mask = tpos < lengths
    neg_inf = jnp.float32(-jnp.inf)
    temp = temp_ref[0, 0]
    logits = jnp.where(mask, scores * temp, neg_inf)

    m = jnp.max(logits, axis=1, keepdims=True)
    e = jnp.exp(logits - m)
    probs = e / jnp.sum(e, axis=1, keepdims=True)

    work = probs
    selected = jnp.zeros((B, T), jnp.bool_)
    for _ in range(TOP_K):
        cur = jnp.max(work, axis=1, keepdims=True)
        idx = jnp.min(jnp.where(work == cur, tpos, T), axis=1, keepdims=True)
        onehot = tpos == idx
        selected = jnp.logical_or(selected, onehot)
        work = jnp.where(onehot, -1.0, work)

    tv = jnp.where(selected, probs, 0.0)
    vsum = jnp.sum(tv, axis=1, keepdims=True)
    att = tv / jnp.maximum(vsum, 1e-12)
    mask_f = mask.astype(jnp.float32)
    uniform = mask_f / (jnp.sum(mask_f, axis=1, keepdims=True) + 1e-08)
    att = jnp.where(vsum > 1e-08, att, uniform)

    seq_f = jnp.sum(xf * att[:, :, None], axis=1)
    seq_b = jnp.sum(xb * att[:, :, None], axis=1)
    out = jnp.dot(seq_f, wto_ref[:HID, :], preferred_element_type=jnp.float32)
    out = out + jnp.dot(seq_b, wto_ref[HID:, :],
                        preferred_element_type=jnp.float32)
    out_ref[...] = out + bto_ref[...]


def _heads(out_f, out_b, w1f, w1b, b1, w2, lengths_col, inv_temp, wto, bto):
    return pl.pallas_call(
        _head_kernel,
        in_specs=[
            pl.BlockSpec((B, T, HID), lambda: (0, 0, 0)),
            pl.BlockSpec((B, T, HID), lambda: (0, 0, 0)),
            pl.BlockSpec((HID, 64), lambda: (0, 0)),
            pl.BlockSpec((HID, 64), lambda: (0, 0)),
            pl.BlockSpec((1, 64), lambda: (0, 0)),
            pl.BlockSpec((1, 1, 64), lambda: (0, 0, 0)),
            pl.BlockSpec((B, 1), lambda: (0, 0)),
            pl.BlockSpec(memory_space=pltpu.SMEM),
            pl.BlockSpec((2 * HID, 32), lambda: (0, 0)),
            pl.BlockSpec((1, 32), lambda: (0, 0)),
        ],
        out_specs=pl.BlockSpec((B, 32), lambda: (0, 0)),
        out_shape=jax.ShapeDtypeStruct((B, 32), jnp.float32),
    )(out_f, out_b, w1f, w1b, b1, w2, lengths_col, inv_temp, wto, bto)


def kernel(frames, params, lengths):
    x2d = frames.reshape(B * T, CHW)

    wenc_t = params['W_enc'].T
    benc = params['b_enc'].reshape(1, FEAT)
    pf, pb = params['gru_fwd'], params['gru_bwd']
    wcat = jnp.concatenate([pf['W_ih'].T, pb['W_ih'].T], axis=1)
    bcat = jnp.concatenate([pf['b_ih'], pb['b_ih']]).reshape(1, 6 * HID)

    gcat = _encode(x2d, wenc_t, benc, wcat, bcat)
    gcat3 = gcat.reshape(B, T, 6 * HID)

    zero = jnp.zeros((HID, 3 * HID), jnp.float32)
    whh_big = jnp.concatenate([
        jnp.concatenate([pf['W_hh'].T, zero], axis=1),
        jnp.concatenate([zero, pb['W_hh'].T], axis=1),
    ], axis=0)
    bhh = jnp.concatenate([pf['b_hh'], pb['b_hh']]).reshape(1, 6 * HID)
    lengths_col = lengths.reshape(B, 1).astype(jnp.int32)

    out_f, out_b = _gru(gcat3, whh_big, bhh, lengths_col)

    w1 = params['W1'].T
    w1f, w1b = w1[:HID, :], w1[HID:, :]
    temp = jnp.clip(params['temperature'], 0.001, 10.0)
    inv_temp = (1.0 / temp).reshape(1, 1)
    b1 = params['b1'].reshape(1, 64)
    w2 = params['W2'].reshape(1, 1, 64)
    wto = jnp.concatenate([params['Wt'].T, params['Wo'].T], axis=1)
    wto = jnp.pad(wto, ((0, 0), (0, 32 - 21)))
    bto = jnp.pad(jnp.concatenate([params['bt'], params['bo']]),
                  (0, 32 - 21)).reshape(1, 32)

    out = _heads(out_f, out_b, w1f, w1b, b1, w2, lengths_col,
                 inv_temp, wto, bto)
    return out[:, :11], out[:, 11:21]


# speedup vs baseline: 3.6271x; 1.0539x over previous
import jax
import jax.numpy as jnp
from jax.experimental import pallas as pl
from jax.experimental.pallas import tpu as pltpu

B, T, C, H, W = 32, 256, 3, 32, 32
FEAT = 512
HID = 128
TOP_K = 8
CHW = C * H * W

_DN_T = (((1,), (1,)), ((), ()))


def _dot_t(a, b):
    return jax.lax.dot_general(a, b, _DN_T,
                               preferred_element_type=jnp.float32)


ROWS_BLK = 1024


def _enc_kernel(x_ref, wenc_ref, benc_ref, wf_ref, wb_ref, bf_ref, bb_ref,
                out_ref):
    x = x_ref[...]
    feats = _dot_t(x, wenc_ref[...]) + benc_ref[...]
    out_ref[:, :3 * HID] = _dot_t(feats, wf_ref[...]) + bf_ref[...]
    out_ref[:, 3 * HID:] = _dot_t(feats, wb_ref[...]) + bb_ref[...]


def _encode(x2d, wenc, benc, wf, wb, bf, bb):
    n_rows = x2d.shape[0]
    grid = (n_rows // ROWS_BLK,)
    return pl.pallas_call(
        _enc_kernel,
        grid=grid,
        in_specs=[
            pl.BlockSpec((ROWS_BLK, CHW), lambda k: (k, 0)),
            pl.BlockSpec((FEAT, CHW), lambda k: (0, 0)),
            pl.BlockSpec((1, FEAT), lambda k: (0, 0)),
            pl.BlockSpec((3 * HID, FEAT), lambda k: (0, 0)),
            pl.BlockSpec((3 * HID, FEAT), lambda k: (0, 0)),
            pl.BlockSpec((1, 3 * HID), lambda k: (0, 0)),
            pl.BlockSpec((1, 3 * HID), lambda k: (0, 0)),
        ],
        out_specs=pl.BlockSpec((ROWS_BLK, 6 * HID), lambda k: (k, 0)),
        out_shape=jax.ShapeDtypeStruct((n_rows, 6 * HID), jnp.float32),
    )(x2d, wenc, benc, wf, wb, bf, bb)


SEQ_BLK = 32


def _gru_kernel(gf_ref, gb_ref, whf_ref, whb_ref, bhf_ref, bhb_ref, len_ref,
                outf_ref, outb_ref, h_ref):
    k = pl.program_id(0)

    @pl.when(k == 0)
    def _():
        h_ref[...] = jnp.zeros_like(h_ref)

    lengths = len_ref[...]

    def step(i, _):
        h = h_ref[...]
        gi_f = gf_ref[:, i, :]
        gi_b = gb_ref[:, SEQ_BLK - 1 - i, :]
        gh_f = _dot_t(h[:, :HID], whf_ref[...]) + bhf_ref[...]
        gh_b = _dot_t(h[:, HID:], whb_ref[...]) + bhb_ref[...]

        t_f = k * SEQ_BLK + i
        t_b = T - 1 - t_f

        def gru_dir(gi, gh_d, h_d, t):
            r = jax.nn.sigmoid(gi[:, :HID] + gh_d[:, :HID])
            z = jax.nn.sigmoid(gi[:, HID:2 * HID] + gh_d[:, HID:2 * HID])
            n = jnp.tanh(gi[:, 2 * HID:] + r * gh_d[:, 2 * HID:])
            h_new = (1.0 - z) * n + z * h_d
            valid = lengths > t
            return jnp.where(valid, h_new, h_d)

        h_f = gru_dir(gi_f, gh_f, h[:, :HID], t_f)
        h_b = gru_dir(gi_b, gh_b, h[:, HID:], t_b)
        outf_ref[:, i, :] = h_f
        outb_ref[:, SEQ_BLK - 1 - i, :] = h_b
        h_ref[...] = jnp.concatenate([h_f, h_b], axis=1)
        return 0

    jax.lax.fori_loop(0, SEQ_BLK, step, 0, unroll=True)


def _gru(gcat3, whf, whb, bhf, bhb, lengths_col):
    nb = T // SEQ_BLK
    out = pl.pallas_call(
        _gru_kernel,
        grid=(nb,),
        in_specs=[
            pl.BlockSpec((B, SEQ_BLK, 3 * HID), lambda k: (0, k, 0)),
            pl.BlockSpec((B, SEQ_BLK, 3 * HID), lambda k, nb=nb: (0, nb - 1 - k, 1)),
            pl.BlockSpec((3 * HID, HID), lambda k: (0, 0)),
            pl.BlockSpec((3 * HID, HID), lambda k: (0, 0)),
            pl.BlockSpec((1, 3 * HID), lambda k: (0, 0)),
            pl.BlockSpec((1, 3 * HID), lambda k: (0, 0)),
            pl.BlockSpec((B, 1), lambda k: (0, 0)),
        ],
        out_specs=[
            pl.BlockSpec((B, SEQ_BLK, HID), lambda k: (0, k, 0)),
            pl.BlockSpec((B, SEQ_BLK, HID), lambda k, nb=nb: (0, nb - 1 - k, 0)),
        ],
        out_shape=[
            jax.ShapeDtypeStruct((B, T, HID), jnp.float32),
            jax.ShapeDtypeStruct((B, T, HID), jnp.float32),
        ],
        scratch_shapes=[pltpu.VMEM((B, 2 * HID), jnp.float32)],
        compiler_params=pltpu.CompilerParams(
            dimension_semantics=("arbitrary",)),
    )(gcat3, gcat3, whf, whb, bhf, bhb, lengths_col)
    return out


def _head_kernel(xf_ref, xb_ref, w1_ref, b1_ref, w2_ref, b2_ref,
                 len_ref, temp_ref, wt_ref, wo_ref, bt_ref, bo_ref,
                 outt_ref, outo_ref):
    xf = xf_ref[...]
    xb = xb_ref[...]
    w1 = w1_ref[...]
    h1 = _dot_t(xf.reshape(B * T, HID), w1[:, :HID])
    h1 = h1 + _dot_t(xb.reshape(B * T, HID), w1[:, HID:])
    h1 = jax.nn.relu(h1 + b1_ref[...])
    scores = jnp.sum(h1.reshape(B, T, 64) * w2_ref[...], axis=2)

    lengths = len_ref[...]
    tpos = jax.lax.broadcasted_iota(jnp.int32, (B, T), 1)
    mask = tpos < lengths
    neg_inf = jnp.float32(-jnp.inf)
    temp = jnp.clip(temp_ref[0, 0], 0.001, 10.0)
    logits = jnp.where(mask, scores * (1.0 / temp), neg_inf)

    m = jnp.max(logits, axis=1, keepdims=True)
    e = jnp.exp(logits - m)
    probs = e / jnp.sum(e, axis=1, keepdims=True)

    work = probs
    selected = jnp.zeros((B, T), jnp.bool_)
    for _ in range(TOP_K):
        cur = jnp.max(work, axis=1, keepdims=True)
        idx = jnp.min(jnp.where(work == cur, tpos, T), axis=1, keepdims=True)
        onehot = tpos == idx
        selected = jnp.logical_or(selected, onehot)
        work = jnp.where(onehot, -1.0, work)

    tv = jnp.where(selected, probs, 0.0)
    vsum = jnp.sum(tv, axis=1, keepdims=True)
    att = tv / jnp.maximum(vsum, 1e-12)
    mask_f = mask.astype(jnp.float32)
    uniform = mask_f / (jnp.sum(mask_f, axis=1, keepdims=True) + 1e-08)
    att = jnp.where(vsum > 1e-08, att, uniform)

    seq_f = jnp.sum(xf * att[:, :, None], axis=1)
    seq_b = jnp.sum(xb * att[:, :, None], axis=1)
    wt = wt_ref[...]
    wo = wo_ref[...]
    outt_ref[...] = (_dot_t(seq_f, wt[:, :HID]) + _dot_t(seq_b, wt[:, HID:])
                     + bt_ref[...])
    outo_ref[...] = (_dot_t(seq_f, wo[:, :HID]) + _dot_t(seq_b, wo[:, HID:])
                     + bo_ref[...])


def _heads(out_f, out_b, w1, b1, w2, b2, lengths_col, temp, wt, wo, bt, bo):
    full = lambda s: pl.BlockSpec(s, lambda *a: tuple(0 for _ in s))
    return pl.pallas_call(
        _head_kernel,
        in_specs=[
            full((B, T, HID)),
            full((B, T, HID)),
            full((64, 2 * HID)),
            full((1, 64)),
            full((1, 1, 64)),
            full((1, 1)),
            full((B, 1)),
            pl.BlockSpec(memory_space=pltpu.SMEM),
            full((11, 2 * HID)),
            full((10, 2 * HID)),
            full((1, 11)),
            full((1, 10)),
        ],
        out_specs=[
            full((B, 11)),
            full((B, 10)),
        ],
        out_shape=[
            jax.ShapeDtypeStruct((B, 11), jnp.float32),
            jax.ShapeDtypeStruct((B, 10), jnp.float32),
        ],
    )(out_f, out_b, w1, b1, w2, b2, lengths_col, temp, wt, wo, bt, bo)


def kernel(frames, params, lengths):
    x2d = frames.reshape(B * T, CHW)
    pf, pb = params['gru_fwd'], params['gru_bwd']

    gcat = _encode(x2d, params['W_enc'], params['b_enc'].reshape(1, FEAT),
                   pf['W_ih'], pb['W_ih'],
                   pf['b_ih'].reshape(1, 3 * HID),
                   pb['b_ih'].reshape(1, 3 * HID))
    gcat3 = gcat.reshape(B, T, 6 * HID)

    lengths_col = lengths.reshape(B, 1)
    out_f, out_b = _gru(gcat3, pf['W_hh'], pb['W_hh'],
                        pf['b_hh'].reshape(1, 3 * HID),
                        pb['b_hh'].reshape(1, 3 * HID), lengths_col)

    tens, ones = _heads(
        out_f, out_b, params['W1'], params['b1'].reshape(1, 64),
        params['W2'].reshape(1, 1, 64), params['b2'].reshape(1, 1),
        lengths_col, params['temperature'].reshape(1, 1),
        params['Wt'], params['Wo'],
        params['bt'].reshape(1, 11), params['bo'].reshape(1, 10))
    return tens, ones


# speedup vs baseline: 9.0584x; 2.4974x over previous
import jax
import jax.numpy as jnp
from jax.experimental import pallas as pl
from jax.experimental.pallas import tpu as pltpu

B, T, C, H, W = 32, 256, 3, 32, 32
FEAT = 512
HID = 128
TOP_K = 8
CHW = C * H * W

_DN_T = (((1,), (1,)), ((), ()))


def _dot_t(a, b):
    return jax.lax.dot_general(a, b, _DN_T,
                               preferred_element_type=jnp.float32)


_DN_LT = (((0,), (1,)), ((), ()))

B_BLK = 2


def _enc_kernel(x_ref, wenc_ref, benc_ref, wf_ref, wb_ref, bf_ref, bb_ref,
                out_ref):
    for j in range(B_BLK):
        feats = jax.lax.dot_general(x_ref[j], wenc_ref[...], _DN_LT,
                                    preferred_element_type=jnp.float32)
        feats = feats + benc_ref[...]
        out_ref[j, :, :3 * HID] = _dot_t(feats, wf_ref[...]) + bf_ref[...]
        out_ref[j, :, 3 * HID:] = _dot_t(feats, wb_ref[...]) + bb_ref[...]


def _encode(xt, wenc, benc, wf, wb, bf, bb):
    return pl.pallas_call(
        _enc_kernel,
        grid=(B // B_BLK,),
        in_specs=[
            pl.BlockSpec((B_BLK, CHW, T), lambda k: (k, 0, 0)),
            pl.BlockSpec((FEAT, CHW), lambda k: (0, 0)),
            pl.BlockSpec((1, FEAT), lambda k: (0, 0)),
            pl.BlockSpec((3 * HID, FEAT), lambda k: (0, 0)),
            pl.BlockSpec((3 * HID, FEAT), lambda k: (0, 0)),
            pl.BlockSpec((1, 3 * HID), lambda k: (0, 0)),
            pl.BlockSpec((1, 3 * HID), lambda k: (0, 0)),
        ],
        out_specs=pl.BlockSpec((B_BLK, T, 6 * HID), lambda k: (k, 0, 0)),
        out_shape=jax.ShapeDtypeStruct((B, T, 6 * HID), jnp.float32),
    )(xt, wenc, benc, wf, wb, bf, bb)


SEQ_BLK = 32


def _gru_kernel(gf_ref, gb_ref, whf_ref, whb_ref, bhf_ref, bhb_ref, len_ref,
                outf_ref, outb_ref, h_ref):
    k = pl.program_id(0)

    @pl.when(k == 0)
    def _():
        h_ref[...] = jnp.zeros_like(h_ref)

    lengths = len_ref[...]

    def step(i, _):
        h = h_ref[...]
        gi_f = gf_ref[:, i, :]
        gi_b = gb_ref[:, SEQ_BLK - 1 - i, :]
        gh_f = _dot_t(h[:, :HID], whf_ref[...]) + bhf_ref[...]
        gh_b = _dot_t(h[:, HID:], whb_ref[...]) + bhb_ref[...]

        t_f = k * SEQ_BLK + i
        t_b = T - 1 - t_f

        def gru_dir(gi, gh_d, h_d, t):
            r = jax.nn.sigmoid(gi[:, :HID] + gh_d[:, :HID])
            z = jax.nn.sigmoid(gi[:, HID:2 * HID] + gh_d[:, HID:2 * HID])
            n = jnp.tanh(gi[:, 2 * HID:] + r * gh_d[:, 2 * HID:])
            h_new = (1.0 - z) * n + z * h_d
            valid = lengths > t
            return jnp.where(valid, h_new, h_d)

        h_f = gru_dir(gi_f, gh_f, h[:, :HID], t_f)
        h_b = gru_dir(gi_b, gh_b, h[:, HID:], t_b)
        outf_ref[:, i, :] = h_f
        outb_ref[:, SEQ_BLK - 1 - i, :] = h_b
        h_ref[...] = jnp.concatenate([h_f, h_b], axis=1)
        return 0

    jax.lax.fori_loop(0, SEQ_BLK, step, 0, unroll=True)


def _gru(gcat3, whf, whb, bhf, bhb, lengths_col):
    nb = T // SEQ_BLK
    out = pl.pallas_call(
        _gru_kernel,
        grid=(nb,),
        in_specs=[
            pl.BlockSpec((B, SEQ_BLK, 3 * HID), lambda k: (0, k, 0)),
            pl.BlockSpec((B, SEQ_BLK, 3 * HID), lambda k, nb=nb: (0, nb - 1 - k, 1)),
            pl.BlockSpec((3 * HID, HID), lambda k: (0, 0)),
            pl.BlockSpec((3 * HID, HID), lambda k: (0, 0)),
            pl.BlockSpec((1, 3 * HID), lambda k: (0, 0)),
            pl.BlockSpec((1, 3 * HID), lambda k: (0, 0)),
            pl.BlockSpec((B, 1), lambda k: (0, 0)),
        ],
        out_specs=[
            pl.BlockSpec((B, SEQ_BLK, HID), lambda k: (0, k, 0)),
            pl.BlockSpec((B, SEQ_BLK, HID), lambda k, nb=nb: (0, nb - 1 - k, 0)),
        ],
        out_shape=[
            jax.ShapeDtypeStruct((B, T, HID), jnp.float32),
            jax.ShapeDtypeStruct((B, T, HID), jnp.float32),
        ],
        scratch_shapes=[pltpu.VMEM((B, 2 * HID), jnp.float32)],
        compiler_params=pltpu.CompilerParams(
            dimension_semantics=("arbitrary",)),
    )(gcat3, gcat3, whf, whb, bhf, bhb, lengths_col)
    return out


def _head_kernel(xf_ref, xb_ref, w1_ref, b1_ref, w2_ref, b2_ref,
                 len_ref, temp_ref, wt_ref, wo_ref, bt_ref, bo_ref,
                 outt_ref, outo_ref):
    xf = xf_ref[...]
    xb = xb_ref[...]
    w1 = w1_ref[...]
    h1 = _dot_t(xf.reshape(B * T, HID), w1[:, :HID])
    h1 = h1 + _dot_t(xb.reshape(B * T, HID), w1[:, HID:])
    h1 = jax.nn.relu(h1 + b1_ref[...])
    scores = jnp.sum(h1.reshape(B, T, 64) * w2_ref[...], axis=2)

    lengths = len_ref[...]
    tpos = jax.lax.broadcasted_iota(jnp.int32, (B, T), 1)
    mask = tpos < lengths
    neg_inf = jnp.float32(-jnp.inf)
    temp = jnp.clip(temp_ref[0, 0], 0.001, 10.0)
    logits = jnp.where(mask, scores * (1.0 / temp), neg_inf)

    m = jnp.max(logits, axis=1, keepdims=True)
    e = jnp.exp(logits - m)
    probs = e / jnp.sum(e, axis=1, keepdims=True)

    work = probs
    selected = jnp.zeros((B, T), jnp.bool_)
    for _ in range(TOP_K):
        cur = jnp.max(work, axis=1, keepdims=True)
        idx = jnp.min(jnp.where(work == cur, tpos, T), axis=1, keepdims=True)
        onehot = tpos == idx
        selected = jnp.logical_or(selected, onehot)
        work = jnp.where(onehot, -1.0, work)

    tv = jnp.where(selected, probs, 0.0)
    vsum = jnp.sum(tv, axis=1, keepdims=True)
    att = tv / jnp.maximum(vsum, 1e-12)
    mask_f = mask.astype(jnp.float32)
    uniform = mask_f / (jnp.sum(mask_f, axis=1, keepdims=True) + 1e-08)
    att = jnp.where(vsum > 1e-08, att, uniform)

    seq_f = jnp.sum(xf * att[:, :, None], axis=1)
    seq_b = jnp.sum(xb * att[:, :, None], axis=1)
    wt = wt_ref[...]
    wo = wo_ref[...]
    outt_ref[...] = (_dot_t(seq_f, wt[:, :HID]) + _dot_t(seq_b, wt[:, HID:])
                     + bt_ref[...])
    outo_ref[...] = (_dot_t(seq_f, wo[:, :HID]) + _dot_t(seq_b, wo[:, HID:])
                     + bo_ref[...])


def _heads(out_f, out_b, w1, b1, w2, b2, lengths_col, temp, wt, wo, bt, bo):
    full = lambda s: pl.BlockSpec(s, lambda *a: tuple(0 for _ in s))
    return pl.pallas_call(
        _head_kernel,
        in_specs=[
            full((B, T, HID)),
            full((B, T, HID)),
            full((64, 2 * HID)),
            full((1, 64)),
            full((1, 1, 64)),
            full((1, 1)),
            full((B, 1)),
            pl.BlockSpec(memory_space=pltpu.SMEM),
            full((11, 2 * HID)),
            full((10, 2 * HID)),
            full((1, 11)),
            full((1, 10)),
        ],
        out_specs=[
            full((B, 11)),
            full((B, 10)),
        ],
        out_shape=[
            jax.ShapeDtypeStruct((B, 11), jnp.float32),
            jax.ShapeDtypeStruct((B, 10), jnp.float32),
        ],
    )(out_f, out_b, w1, b1, w2, b2, lengths_col, temp, wt, wo, bt, bo)


def kernel(frames, params, lengths):
    xt = jnp.transpose(frames, (0, 2, 3, 4, 1)).reshape(B, CHW, T)
    pf, pb = params['gru_fwd'], params['gru_bwd']

    gcat3 = _encode(xt, params['W_enc'], params['b_enc'].reshape(1, FEAT),
                    pf['W_ih'], pb['W_ih'],
                    pf['b_ih'].reshape(1, 3 * HID),
                    pb['b_ih'].reshape(1, 3 * HID))

    lengths_col = lengths.reshape(B, 1)
    out_f, out_b = _gru(gcat3, pf['W_hh'], pb['W_hh'],
                        pf['b_hh'].reshape(1, 3 * HID),
                        pb['b_hh'].reshape(1, 3 * HID), lengths_col)

    tens, ones = _heads(
        out_f, out_b, params['W1'], params['b1'].reshape(1, 64),
        params['W2'].reshape(1, 1, 64), params['b2'].reshape(1, 1),
        lengths_col, params['temperature'].reshape(1, 1),
        params['Wt'], params['Wo'],
        params['bt'].reshape(1, 11), params['bo'].reshape(1, 10))
    return tens, ones
